# baseline (speedup 1.0000x reference)
import functools

import jax
import jax.numpy as jnp
from jax import lax
from jax.experimental import pallas as pl
from jax.experimental.pallas import tpu as pltpu
from jax.experimental.pallas import tpu_sc as plsc

EMA = 0.7
LAM = 3.0
CLIP_LO = 0.0001
CLIP_HI = 1.0 - 0.0001

NC = 2
NS = 16
NW = NC * NS
L = 16
CH = 128
BS = 512


def _wid():
    return lax.axis_index("s") * NC + lax.axis_index("c")


def _sc_a_body(B, C, vsp, idx_hbm, lab_hbm, outflat_hbm, q_hbm,
               t_hbm, qrows_hbm, pick_hbm,
               idx_all, t_loc, rows_v, lab_v, fidx_v, pick_v, acc_v,
               sem, sem2, sem3):
    wid = _wid()
    rpw = B // NW
    nk = rpw // CH
    lo = wid * vsp
    base = wid * rpw

    pltpu.sync_copy(idx_hbm, idx_all)
    pltpu.sync_copy(lab_hbm.at[pl.ds(base, rpw)], lab_v)

    qc = [pltpu.async_copy(
        q_hbm.at[idx_all.at[pl.ds(base + k * CH, CH)]],
        rows_v.at[pl.ds(k * CH, CH)], sem) for k in range(nk)]

    iota = lax.iota(jnp.int32, L)
    for k in range(rpw // L):
        row = (base + k * L) + iota
        fidx_v[pl.ds(k * L, L)] = row * C + lab_v[pl.ds(k * L, L)]
    pc = [pltpu.async_copy(
        outflat_hbm.at[fidx_v.at[pl.ds(k * CH, CH)]],
        pick_v.at[pl.ds(k * CH, CH)], sem3) for k in range(nk)]

    def step(k, carry):
        for u in range(8):
            kk = k * 8 + u
            iv = idx_all[pl.ds(kk * L, L)]
            jv = kk * L + iota
            owned = (iv >= lo) & (iv < lo + vsp)
            plsc.store_scatter(t_loc, [iv - lo], jv, mask=owned)
        return carry
    lax.fori_loop(0, B // L // 8, step, 0)
    pltpu.sync_copy(t_loc, t_hbm.at[pl.ds(lo, vsp)])

    for cp in qc:
        cp.wait()
    pltpu.sync_copy(rows_v, qrows_hbm.at[pl.ds(base, rpw)])

    for cp in pc:
        cp.wait()
    acc = pick_v[pl.ds(0, L)]
    for k in range(1, rpw // L):
        acc = acc + pick_v[pl.ds(k * L, L)]
    acc_v[...] = acc
    pltpu.sync_copy(acc_v, pick_hbm.at[wid])


def _sc_a(index, label, outflat, Q, vsp):
    B = index.shape[0]
    N, C = Q.shape
    rpw = B // NW
    mesh = plsc.VectorSubcoreMesh(core_axis_name="c", subcore_axis_name="s",
                                  num_cores=NC, num_subcores=NS)
    return pl.kernel(
        functools.partial(_sc_a_body, B, C, vsp),
        out_type=[
            jax.ShapeDtypeStruct((NW * vsp,), jnp.int32),
            jax.ShapeDtypeStruct((B, C), jnp.float32),
            jax.ShapeDtypeStruct((NW, L), jnp.float32),
        ],
        mesh=mesh,
        compiler_params=pltpu.CompilerParams(needs_layout_passes=False),
        scratch_types=[
            pltpu.VMEM((B,), jnp.int32),
            pltpu.VMEM((vsp,), jnp.int32),
            pltpu.VMEM((rpw, C), jnp.float32),
            pltpu.VMEM((rpw,), jnp.int32),
            pltpu.VMEM((rpw,), jnp.int32),
            pltpu.VMEM((rpw,), jnp.float32),
            pltpu.VMEM((L,), jnp.float32),
            pltpu.SemaphoreType.DMA,
            pltpu.SemaphoreType.DMA,
            pltpu.SemaphoreType.DMA,
        ],
    )(index, label, outflat, Q)


def _sc_b_body(B, idx3_hbm, t_hbm, out_hbm, outg_hbm, idx_v, jl_v, rows_v,
               sem, sem2):
    wid = _wid()
    rpw = B // NW
    nk = rpw // CH

    pltpu.sync_copy(idx3_hbm.at[wid], idx_v)
    jc = [pltpu.async_copy(t_hbm.at[idx_v.at[k]], jl_v.at[k], sem2)
          for k in range(nk)]
    for cp in jc:
        cp.wait()
    dc = [pltpu.async_copy(out_hbm.at[jl_v.at[k]],
                           rows_v.at[pl.ds(k * CH, CH)], sem)
          for k in range(nk)]
    for cp in dc:
        cp.wait()
    pltpu.sync_copy(rows_v, outg_hbm.at[pl.ds(wid * rpw, rpw)])


def _sc_b(index3, t, output):
    B, C = output.shape
    rpw = B // NW
    mesh = plsc.VectorSubcoreMesh(core_axis_name="c", subcore_axis_name="s",
                                  num_cores=NC, num_subcores=NS)
    return pl.kernel(
        functools.partial(_sc_b_body, B),
        out_type=jax.ShapeDtypeStruct((B, C), jnp.float32),
        mesh=mesh,
        scratch_types=[
            pltpu.VMEM((rpw // CH, CH), jnp.int32),
            pltpu.VMEM((rpw // CH, CH), jnp.int32),
            pltpu.VMEM((rpw, C), jnp.float32),
            pltpu.SemaphoreType.DMA,
            pltpu.SemaphoreType.DMA,
        ],
    )(index3, t, output)


def _prob_body(out_ref, yp_ref, logs_ref, acc_ref):
    x = out_ref[...]
    ex = jnp.exp(x)
    ones = jnp.ones((x.shape[1], 1), jnp.float32)
    s_col = jnp.dot(ex, ones, preferred_element_type=jnp.float32)
    yp_ref[...] = jnp.clip(ex * (1.0 / s_col), CLIP_LO, CLIP_HI)
    ls = jnp.log(s_col)
    onesr = jnp.ones((1, ls.shape[0]), jnp.float32)
    part = jnp.dot(onesr, ls, preferred_element_type=jnp.float32)

    @pl.when(pl.program_id(0) == 0)
    def _():
        acc_ref[...] = jnp.zeros((1, 1), jnp.float32)
    acc_ref[...] += part

    @pl.when(pl.program_id(0) == pl.num_programs(0) - 1)
    def _():
        logs_ref[...] = acc_ref[...]


def _tc_prob(output):
    B, C = output.shape
    G = B // BS
    return pl.pallas_call(
        _prob_body,
        grid=(G,),
        in_specs=[pl.BlockSpec((BS, C), lambda i: (i, 0))],
        out_specs=[
            pl.BlockSpec((BS, C), lambda i: (i, 0)),
            pl.BlockSpec((1, 1), lambda i: (0, 0)),
        ],
        out_shape=[
            jax.ShapeDtypeStruct((B, C), jnp.float32),
            jax.ShapeDtypeStruct((1, 1), jnp.float32),
        ],
        scratch_shapes=[pltpu.VMEM((1, 1), jnp.float32)],
    )(output)


def _loss_body(B, qr_ref, og_ref, yp_ref, pp_ref, logs_ref, res_ref, acc_ref):
    xg = og_ref[...]
    eg = jnp.exp(xg)
    ones = jnp.ones((xg.shape[1], 1), jnp.float32)
    sg_col = jnp.dot(eg, ones, preferred_element_type=jnp.float32)
    ypg = jnp.clip(eg * (1.0 / sg_col), CLIP_LO, CLIP_HI)
    spg_col = jnp.dot(ypg, ones, preferred_element_type=jnp.float32)
    ydg = ypg * (1.0 / spg_col)

    p = (EMA * qr_ref[...] + (1.0 - EMA) * ydg) * yp_ref[...]
    inner_col = jnp.dot(p, ones, preferred_element_type=jnp.float32)
    w_col = jnp.log(1.0 - inner_col)
    onesr = jnp.ones((1, w_col.shape[0]), jnp.float32)
    part = jnp.dot(onesr, w_col, preferred_element_type=jnp.float32)

    @pl.when(pl.program_id(0) == 0)
    def _():
        acc_ref[...] = jnp.zeros((1, 1), jnp.float32)
    acc_ref[...] += part

    @pl.when(pl.program_id(0) == pl.num_programs(0) - 1)
    def _():
        res_ref[...] = (LAM * acc_ref[...] + logs_ref[...]
                        - jnp.sum(pp_ref[...])) / B


def _tc_loss(qrows, outg, ypred, pick, logs):
    B, C = outg.shape
    G = B // BS
    return pl.pallas_call(
        functools.partial(_loss_body, B),
        grid=(G,),
        in_specs=[
            pl.BlockSpec((BS, C), lambda i: (i, 0)),
            pl.BlockSpec((BS, C), lambda i: (i, 0)),
            pl.BlockSpec((BS, C), lambda i: (i, 0)),
            pl.BlockSpec((NW, L), lambda i: (0, 0)),
            pl.BlockSpec((1, 1), lambda i: (0, 0)),
        ],
        out_specs=pl.BlockSpec((1, 1), lambda i: (0, 0)),
        out_shape=jax.ShapeDtypeStruct((1, 1), jnp.float32),
        scratch_shapes=[pltpu.VMEM((1, 1), jnp.float32)],
    )(qrows, outg, ypred, pick, logs)


def kernel(index, output, label, Q):
    B, C = output.shape
    N = Q.shape[0]
    rpw = B // NW
    vsp = ((N + NW - 1) // NW + 7) // 8 * 8

    idx = index.astype(jnp.int32)
    index3 = idx.reshape(NW, rpw // CH, CH)
    lab = label.astype(jnp.int32)
    outflat = output.reshape(B * C)

    ypred, logs = _tc_prob(output)
    t, qrows, pick = _sc_a(idx, lab, outflat, Q, vsp)
    outg = _sc_b(index3, t, output)
    res = _tc_loss(qrows, outg, ypred, pick, logs)
    return res[0, 0]


# speedup vs baseline: 1.1054x; 1.1054x over previous
import functools

import jax
import jax.numpy as jnp
from jax import lax
from jax.experimental import pallas as pl
from jax.experimental.pallas import tpu as pltpu
from jax.experimental.pallas import tpu_sc as plsc

EMA = 0.7
LAM = 3.0
CLIP_LO = 0.0001
CLIP_HI = 1.0 - 0.0001

NC = 2
NS = 16
NW = NC * NS
L = 16
CH = 128
BS = 512


def _wid():
    return lax.axis_index("s") * NC + lax.axis_index("c")


def _sc_a_body(B, C, vsp, idx_hbm, lab_hbm, outflat_hbm, q_hbm,
               t_hbm, qrows_hbm, pick_hbm,
               idx_all, t_loc, rows_v, lab_v, fidx_v, pick_v, acc_v,
               sem, sem2, sem3):
    wid = _wid()
    rpw = B // NW
    nk = rpw // CH
    lo = wid * vsp
    base = wid * rpw

    pltpu.sync_copy(idx_hbm, idx_all)
    pltpu.sync_copy(lab_hbm.at[pl.ds(base, rpw)], lab_v)

    qc = [pltpu.async_copy(
        q_hbm.at[idx_all.at[pl.ds(base + k * CH, CH)]],
        rows_v.at[pl.ds(k * CH, CH)], sem) for k in range(nk)]

    iota = lax.iota(jnp.int32, L)
    for k in range(rpw // L):
        row = (base + k * L) + iota
        fidx_v[pl.ds(k * L, L)] = row * C + lab_v[pl.ds(k * L, L)]
    pc = [pltpu.async_copy(
        outflat_hbm.at[fidx_v.at[pl.ds(k * CH, CH)]],
        pick_v.at[pl.ds(k * CH, CH)], sem3) for k in range(nk)]

    def step(k, carry):
        for u in range(8):
            kk = k * 8 + u
            iv = idx_all[pl.ds(kk * L, L)]
            jv = kk * L + iota
            owned = (iv >= lo) & (iv < lo + vsp)
            plsc.store_scatter(t_loc, [iv - lo], jv, mask=owned)
        return carry
    lax.fori_loop(0, B // L // 8, step, 0)
    pltpu.sync_copy(t_loc, t_hbm.at[pl.ds(lo, vsp)])

    for cp in qc:
        cp.wait()
    pltpu.sync_copy(rows_v, qrows_hbm.at[pl.ds(base, rpw)])

    for cp in pc:
        cp.wait()
    acc = pick_v[pl.ds(0, L)]
    for k in range(1, rpw // L):
        acc = acc + pick_v[pl.ds(k * L, L)]
    acc_v[...] = acc
    pltpu.sync_copy(acc_v, pick_hbm.at[wid])


def _sc_a(index, label, outflat, Q, vsp):
    B = index.shape[0]
    N, C = Q.shape
    rpw = B // NW
    mesh = plsc.VectorSubcoreMesh(core_axis_name="c", subcore_axis_name="s",
                                  num_cores=NC, num_subcores=NS)
    return pl.kernel(
        functools.partial(_sc_a_body, B, C, vsp),
        out_type=[
            jax.ShapeDtypeStruct((NW * vsp,), jnp.int32),
            jax.ShapeDtypeStruct((B, C), jnp.float32),
            jax.ShapeDtypeStruct((NW, L), jnp.float32),
        ],
        mesh=mesh,
        compiler_params=pltpu.CompilerParams(needs_layout_passes=False),
        scratch_types=[
            pltpu.VMEM((B,), jnp.int32),
            pltpu.VMEM((vsp,), jnp.int32),
            pltpu.VMEM((rpw, C), jnp.float32),
            pltpu.VMEM((rpw,), jnp.int32),
            pltpu.VMEM((rpw,), jnp.int32),
            pltpu.VMEM((rpw,), jnp.float32),
            pltpu.VMEM((L,), jnp.float32),
            pltpu.SemaphoreType.DMA,
            pltpu.SemaphoreType.DMA,
            pltpu.SemaphoreType.DMA,
        ],
    )(index, label, outflat, Q)


def _sc_b_body(B, idx3_hbm, t_hbm, yd_hbm, ydl_hbm, idx_v, jl_v, rows_v,
               sem, sem2):
    wid = _wid()
    rpw = B // NW
    nk = rpw // CH

    pltpu.sync_copy(idx3_hbm.at[wid], idx_v)
    jc = [pltpu.async_copy(t_hbm.at[idx_v.at[k]], jl_v.at[k], sem2)
          for k in range(nk)]
    for cp in jc:
        cp.wait()
    dc = [pltpu.async_copy(yd_hbm.at[jl_v.at[k]],
                           rows_v.at[pl.ds(k * CH, CH)], sem)
          for k in range(nk)]
    for cp in dc:
        cp.wait()
    pltpu.sync_copy(rows_v, ydl_hbm.at[pl.ds(wid * rpw, rpw)])


def _sc_b(index3, t, ydet):
    B, C = ydet.shape
    rpw = B // NW
    mesh = plsc.VectorSubcoreMesh(core_axis_name="c", subcore_axis_name="s",
                                  num_cores=NC, num_subcores=NS)
    return pl.kernel(
        functools.partial(_sc_b_body, B),
        out_type=jax.ShapeDtypeStruct((B, C), jnp.float32),
        mesh=mesh,
        scratch_types=[
            pltpu.VMEM((rpw // CH, CH), jnp.int32),
            pltpu.VMEM((rpw // CH, CH), jnp.int32),
            pltpu.VMEM((rpw, C), jnp.float32),
            pltpu.SemaphoreType.DMA,
            pltpu.SemaphoreType.DMA,
        ],
    )(index3, t, ydet)


def _prob_body(out_ref, yd_ref):
    x = out_ref[...]
    ex = jnp.exp(x)
    ones = jnp.ones((x.shape[1], 1), jnp.float32)
    s_col = jnp.dot(ex, ones, preferred_element_type=jnp.float32)
    yp = jnp.clip(ex * (1.0 / s_col), CLIP_LO, CLIP_HI)
    sp_col = jnp.dot(yp, ones, preferred_element_type=jnp.float32)
    yd_ref[...] = yp * (1.0 / sp_col)


def _tc_prob(output):
    B, C = output.shape
    G = B // BS
    return pl.pallas_call(
        _prob_body,
        grid=(G,),
        in_specs=[pl.BlockSpec((BS, C), lambda i: (i, 0))],
        out_specs=pl.BlockSpec((BS, C), lambda i: (i, 0)),
        out_shape=jax.ShapeDtypeStruct((B, C), jnp.float32),
    )(output)


def _loss_body(B, out_ref, qr_ref, ydl_ref, pp_ref, res_ref, acc_ref):
    x = out_ref[...]
    ex = jnp.exp(x)
    ones = jnp.ones((x.shape[1], 1), jnp.float32)
    s_col = jnp.dot(ex, ones, preferred_element_type=jnp.float32)
    yp = jnp.clip(ex * (1.0 / s_col), CLIP_LO, CLIP_HI)

    p = (EMA * qr_ref[...] + (1.0 - EMA) * ydl_ref[...]) * yp
    inner_col = jnp.dot(p, ones, preferred_element_type=jnp.float32)
    w_col = jnp.log(s_col) + LAM * jnp.log(1.0 - inner_col)
    onesr = jnp.ones((1, w_col.shape[0]), jnp.float32)
    part = jnp.dot(onesr, w_col, preferred_element_type=jnp.float32)

    @pl.when(pl.program_id(0) == 0)
    def _():
        acc_ref[...] = jnp.zeros((1, 1), jnp.float32)
    acc_ref[...] += part

    @pl.when(pl.program_id(0) == pl.num_programs(0) - 1)
    def _():
        res_ref[...] = (acc_ref[...] - jnp.sum(pp_ref[...])) / B


def _tc_loss(output, qrows, ydl, pick):
    B, C = output.shape
    G = B // BS
    return pl.pallas_call(
        functools.partial(_loss_body, B),
        grid=(G,),
        in_specs=[
            pl.BlockSpec((BS, C), lambda i: (i, 0)),
            pl.BlockSpec((BS, C), lambda i: (i, 0)),
            pl.BlockSpec((BS, C), lambda i: (i, 0)),
            pl.BlockSpec((NW, L), lambda i: (0, 0)),
        ],
        out_specs=pl.BlockSpec((1, 1), lambda i: (0, 0)),
        out_shape=jax.ShapeDtypeStruct((1, 1), jnp.float32),
        scratch_shapes=[pltpu.VMEM((1, 1), jnp.float32)],
    )(output, qrows, ydl, pick)


def kernel(index, output, label, Q):
    B, C = output.shape
    N = Q.shape[0]
    rpw = B // NW
    vsp = ((N + NW - 1) // NW + 7) // 8 * 8

    idx = index.astype(jnp.int32)
    index3 = idx.reshape(NW, rpw // CH, CH)
    lab = label.astype(jnp.int32)
    outflat = output.reshape(B * C)

    ydet = _tc_prob(output)
    t, qrows, pick = _sc_a(idx, lab, outflat, Q, vsp)
    ydl = _sc_b(index3, t, ydet)
    res = _tc_loss(output, qrows, ydl, pick)
    return res[0, 0]


# speedup vs baseline: 1.3933x; 1.2605x over previous
import functools

import jax
import jax.numpy as jnp
from jax import lax
from jax.experimental import pallas as pl
from jax.experimental.pallas import tpu as pltpu
from jax.experimental.pallas import tpu_sc as plsc

EMA = 0.7
LAM = 3.0
CLIP_LO = 0.0001
CLIP_HI = 1.0 - 0.0001

NC = 2
NS = 16
NW = NC * NS
L = 16
CH = 128
BS = 1024


def _wid():
    return lax.axis_index("s") * NC + lax.axis_index("c")


def _sc_a_body(B, C, vsp, idx_hbm, lab_hbm, outflat_hbm, q_hbm,
               t_hbm, qrows_hbm, pick_hbm,
               idx_all, t_loc, rows_v, lab_v, fidx_v, pick_v, acc_v,
               sem, sem2, sem3):
    wid = _wid()
    rpw = B // NW
    nk = rpw // CH
    lo = wid * vsp
    base = wid * rpw

    pltpu.sync_copy(idx_hbm, idx_all)
    pltpu.sync_copy(lab_hbm.at[pl.ds(base, rpw)], lab_v)

    qc = [pltpu.async_copy(
        q_hbm.at[idx_all.at[pl.ds(base + k * CH, CH)]],
        rows_v.at[pl.ds(k * CH, CH)], sem) for k in range(nk)]

    iota = lax.iota(jnp.int32, L)
    for k in range(rpw // L):
        row = (base + k * L) + iota
        fidx_v[pl.ds(k * L, L)] = row * C + lab_v[pl.ds(k * L, L)]
    pc = [pltpu.async_copy(
        outflat_hbm.at[fidx_v.at[pl.ds(k * CH, CH)]],
        pick_v.at[pl.ds(k * CH, CH)], sem3) for k in range(nk)]

    def step(k, carry):
        for u in range(8):
            kk = k * 8 + u
            iv = idx_all[pl.ds(kk * L, L)]
            jv = kk * L + iota
            owned = (iv >= lo) & (iv < lo + vsp)
            plsc.store_scatter(t_loc, [iv - lo], jv, mask=owned)
        return carry
    lax.fori_loop(0, B // L // 8, step, 0)
    pltpu.sync_copy(t_loc, t_hbm.at[pl.ds(lo, vsp)])

    for cp in qc:
        cp.wait()
    pltpu.sync_copy(rows_v, qrows_hbm.at[pl.ds(base, rpw)])

    for cp in pc:
        cp.wait()
    acc = pick_v[pl.ds(0, L)]
    for k in range(1, rpw // L):
        acc = acc + pick_v[pl.ds(k * L, L)]
    acc_v[...] = acc
    pltpu.sync_copy(acc_v, pick_hbm.at[wid])


def _sc_a(index, label, outflat, Q, vsp):
    B = index.shape[0]
    N, C = Q.shape
    rpw = B // NW
    mesh = plsc.VectorSubcoreMesh(core_axis_name="c", subcore_axis_name="s",
                                  num_cores=NC, num_subcores=NS)
    return pl.kernel(
        functools.partial(_sc_a_body, B, C, vsp),
        out_type=[
            jax.ShapeDtypeStruct((NW * vsp,), jnp.int32),
            jax.ShapeDtypeStruct((B, C), jnp.float32),
            jax.ShapeDtypeStruct((NW, L), jnp.float32),
        ],
        mesh=mesh,
        compiler_params=pltpu.CompilerParams(needs_layout_passes=False),
        scratch_types=[
            pltpu.VMEM((B,), jnp.int32),
            pltpu.VMEM((vsp,), jnp.int32),
            pltpu.VMEM((rpw, C), jnp.float32),
            pltpu.VMEM((rpw,), jnp.int32),
            pltpu.VMEM((rpw,), jnp.int32),
            pltpu.VMEM((rpw,), jnp.float32),
            pltpu.VMEM((L,), jnp.float32),
            pltpu.SemaphoreType.DMA,
            pltpu.SemaphoreType.DMA,
            pltpu.SemaphoreType.DMA,
        ],
    )(index, label, outflat, Q)


def _sc_b_body(B, idx3_hbm, t_hbm, src_hbm, outg_hbm, idx_v, jl_v, rows_v,
               semr, s0, s1, s2, s3):
    wid = _wid()
    rpw = B // NW
    nk = rpw // CH
    sems = [s0, s1, s2, s3]

    pltpu.sync_copy(idx3_hbm.at[wid], idx_v)
    jc = [pltpu.async_copy(t_hbm.at[idx_v.at[k]], jl_v.at[k], sems[k])
          for k in range(nk)]
    dc = []
    for k in range(nk):
        jc[k].wait()
        dc.append(pltpu.async_copy(src_hbm.at[jl_v.at[k]],
                                   rows_v.at[pl.ds(k * CH, CH)], semr))
    for cp in dc:
        cp.wait()
    pltpu.sync_copy(rows_v, outg_hbm.at[pl.ds(wid * rpw, rpw)])


def _sc_b(index3, t, src):
    B, C = src.shape
    rpw = B // NW
    mesh = plsc.VectorSubcoreMesh(core_axis_name="c", subcore_axis_name="s",
                                  num_cores=NC, num_subcores=NS)
    return pl.kernel(
        functools.partial(_sc_b_body, B),
        out_type=jax.ShapeDtypeStruct((B, C), jnp.float32),
        mesh=mesh,
        scratch_types=[
            pltpu.VMEM((rpw // CH, CH), jnp.int32),
            pltpu.VMEM((rpw // CH, CH), jnp.int32),
            pltpu.VMEM((rpw, C), jnp.float32),
            pltpu.SemaphoreType.DMA,
            pltpu.SemaphoreType.DMA,
            pltpu.SemaphoreType.DMA,
            pltpu.SemaphoreType.DMA,
            pltpu.SemaphoreType.DMA,
        ],
    )(index3, t, src)


def _loss_body(B, out_ref, qr_ref, og_ref, pp_ref, res_ref, acc_ref):
    x = out_ref[...]
    ex = jnp.exp(x)
    ones = jnp.ones((x.shape[1], 1), jnp.float32)
    s_col = jnp.dot(ex, ones, preferred_element_type=jnp.float32)
    yp = jnp.clip(ex * (1.0 / s_col), CLIP_LO, CLIP_HI)

    xg = og_ref[...]
    eg = jnp.exp(xg)
    sg_col = jnp.dot(eg, ones, preferred_element_type=jnp.float32)
    ypg = jnp.clip(eg * (1.0 / sg_col), CLIP_LO, CLIP_HI)
    spg_col = jnp.dot(ypg, ones, preferred_element_type=jnp.float32)
    ydg = ypg * (1.0 / spg_col)

    p = (EMA * qr_ref[...] + (1.0 - EMA) * ydg) * yp
    inner_col = jnp.dot(p, ones, preferred_element_type=jnp.float32)
    w_col = jnp.log(s_col) + LAM * jnp.log(1.0 - inner_col)
    onesr = jnp.ones((1, w_col.shape[0]), jnp.float32)
    part = jnp.dot(onesr, w_col, preferred_element_type=jnp.float32)

    @pl.when(pl.program_id(0) == 0)
    def _():
        acc_ref[...] = jnp.zeros((1, 1), jnp.float32)
    acc_ref[...] += part

    @pl.when(pl.program_id(0) == pl.num_programs(0) - 1)
    def _():
        res_ref[...] = (acc_ref[...] - jnp.sum(pp_ref[...])) / B


def _tc_loss(output, qrows, outg, pick):
    B, C = output.shape
    G = B // BS
    return pl.pallas_call(
        functools.partial(_loss_body, B),
        grid=(G,),
        in_specs=[
            pl.BlockSpec((BS, C), lambda i: (i, 0)),
            pl.BlockSpec((BS, C), lambda i: (i, 0)),
            pl.BlockSpec((BS, C), lambda i: (i, 0)),
            pl.BlockSpec((NW, L), lambda i: (0, 0)),
        ],
        out_specs=pl.BlockSpec((1, 1), lambda i: (0, 0)),
        out_shape=jax.ShapeDtypeStruct((1, 1), jnp.float32),
        scratch_shapes=[pltpu.VMEM((1, 1), jnp.float32)],
    )(output, qrows, outg, pick)


def kernel(index, output, label, Q):
    B, C = output.shape
    N = Q.shape[0]
    rpw = B // NW
    vsp = ((N + NW - 1) // NW + 7) // 8 * 8

    idx = index.astype(jnp.int32)
    index3 = idx.reshape(NW, rpw // CH, CH)
    lab = label.astype(jnp.int32)
    outflat = output.reshape(B * C)

    t, qrows, pick = _sc_a(idx, lab, outflat, Q, vsp)
    outg = _sc_b(index3, t, output)
    res = _tc_loss(output, qrows, outg, pick)
    return res[0, 0]


# speedup vs baseline: 1.4863x; 1.0667x over previous
import functools

import jax
import jax.numpy as jnp
from jax import lax
from jax.experimental import pallas as pl
from jax.experimental.pallas import tpu as pltpu
from jax.experimental.pallas import tpu_sc as plsc

EMA = 0.7
LAM = 3.0
CLIP_LO = 0.0001
CLIP_HI = 1.0 - 0.0001

NC = 2
NS = 16
NW = NC * NS
L = 16
CH = 128
BS = 2048


def _wid():
    return lax.axis_index("s") * NC + lax.axis_index("c")


def _sc_a_body(B, C, vsp, idx_hbm, lab_hbm, outflat_hbm, q_hbm,
               t_hbm, qrows_hbm, pick_hbm,
               idx_all, t_loc, rows_v, lab_v, fidx_v, pick_v, acc_v,
               sem, sem2, sem3):
    wid = _wid()
    rpw = B // NW
    nk = rpw // CH
    lo = wid * vsp
    base = wid * rpw

    pltpu.sync_copy(idx_hbm, idx_all)
    pltpu.sync_copy(lab_hbm.at[pl.ds(base, rpw)], lab_v)

    qc = [pltpu.async_copy(
        q_hbm.at[idx_all.at[pl.ds(base + k * CH, CH)]],
        rows_v.at[pl.ds(k * CH, CH)], sem) for k in range(nk)]

    iota = lax.iota(jnp.int32, L)
    for k in range(rpw // L):
        row = (base + k * L) + iota
        fidx_v[pl.ds(k * L, L)] = row * C + lab_v[pl.ds(k * L, L)]
    pc = [pltpu.async_copy(
        outflat_hbm.at[fidx_v.at[pl.ds(k * CH, CH)]],
        pick_v.at[pl.ds(k * CH, CH)], sem3) for k in range(nk)]

    def step(k, carry):
        for u in range(8):
            kk = k * 8 + u
            iv = idx_all[pl.ds(kk * L, L)]
            jv = kk * L + iota
            owned = (iv >= lo) & (iv < lo + vsp)
            plsc.store_scatter(t_loc, [iv - lo], jv, mask=owned)
        return carry
    lax.fori_loop(0, B // L // 8, step, 0)
    pltpu.sync_copy(t_loc, t_hbm.at[pl.ds(lo, vsp)])

    for cp in qc:
        cp.wait()
    pltpu.sync_copy(rows_v, qrows_hbm.at[pl.ds(base, rpw)])

    for cp in pc:
        cp.wait()
    acc = pick_v[pl.ds(0, L)]
    for k in range(1, rpw // L):
        acc = acc + pick_v[pl.ds(k * L, L)]
    acc_v[...] = acc
    pltpu.sync_copy(acc_v, pick_hbm.at[wid])


def _sc_a(index, label, outflat, Q, vsp):
    B = index.shape[0]
    N, C = Q.shape
    rpw = B // NW
    mesh = plsc.VectorSubcoreMesh(core_axis_name="c", subcore_axis_name="s",
                                  num_cores=NC, num_subcores=NS)
    return pl.kernel(
        functools.partial(_sc_a_body, B, C, vsp),
        out_type=[
            jax.ShapeDtypeStruct((NW * vsp,), jnp.int32),
            jax.ShapeDtypeStruct((B, C), jnp.float32),
            jax.ShapeDtypeStruct((NW, L), jnp.float32),
        ],
        mesh=mesh,
        compiler_params=pltpu.CompilerParams(needs_layout_passes=False),
        scratch_types=[
            pltpu.VMEM((B,), jnp.int32),
            pltpu.VMEM((vsp,), jnp.int32),
            pltpu.VMEM((rpw, C), jnp.float32),
            pltpu.VMEM((rpw,), jnp.int32),
            pltpu.VMEM((rpw,), jnp.int32),
            pltpu.VMEM((rpw,), jnp.float32),
            pltpu.VMEM((L,), jnp.float32),
            pltpu.SemaphoreType.DMA,
            pltpu.SemaphoreType.DMA,
            pltpu.SemaphoreType.DMA,
        ],
    )(index, label, outflat, Q)


def _sc_b_body(B, idx3_hbm, t_hbm, src_hbm, outg_hbm, idx_v, jl_v, rows_v,
               semr, s0, s1, s2, s3):
    wid = _wid()
    rpw = B // NW
    nk = rpw // CH
    sems = [s0, s1, s2, s3]

    pltpu.sync_copy(idx3_hbm.at[wid], idx_v)
    jc = [pltpu.async_copy(t_hbm.at[idx_v.at[k]], jl_v.at[k], sems[k])
          for k in range(nk)]
    dc = []
    for k in range(nk):
        jc[k].wait()
        dc.append(pltpu.async_copy(src_hbm.at[jl_v.at[k]],
                                   rows_v.at[pl.ds(k * CH, CH)], semr))
    for cp in dc:
        cp.wait()
    pltpu.sync_copy(rows_v, outg_hbm.at[pl.ds(wid * rpw, rpw)])


def _sc_b(index3, t, src):
    B, C = src.shape
    rpw = B // NW
    mesh = plsc.VectorSubcoreMesh(core_axis_name="c", subcore_axis_name="s",
                                  num_cores=NC, num_subcores=NS)
    return pl.kernel(
        functools.partial(_sc_b_body, B),
        out_type=jax.ShapeDtypeStruct((B, C), jnp.float32),
        mesh=mesh,
        scratch_types=[
            pltpu.VMEM((rpw // CH, CH), jnp.int32),
            pltpu.VMEM((rpw // CH, CH), jnp.int32),
            pltpu.VMEM((rpw, C), jnp.float32),
            pltpu.SemaphoreType.DMA,
            pltpu.SemaphoreType.DMA,
            pltpu.SemaphoreType.DMA,
            pltpu.SemaphoreType.DMA,
            pltpu.SemaphoreType.DMA,
        ],
    )(index3, t, src)


def _loss_body(B, out_ref, qr_ref, og_ref, pp_ref, res_ref, acc_ref):
    x = out_ref[...]
    ex = jnp.exp(x)
    ones = jnp.ones((x.shape[1], 1), jnp.float32)
    s_col = jnp.dot(ex, ones, preferred_element_type=jnp.float32)
    yp = jnp.clip(ex * (1.0 / s_col), CLIP_LO, CLIP_HI)

    xg = og_ref[...]
    eg = jnp.exp(xg)
    sg_col = jnp.dot(eg, ones, preferred_element_type=jnp.float32)
    ypg = jnp.clip(eg * (1.0 / sg_col), CLIP_LO, CLIP_HI)
    spg_col = jnp.dot(ypg, ones, preferred_element_type=jnp.float32)
    ydg = ypg * (1.0 / spg_col)

    p = (EMA * qr_ref[...] + (1.0 - EMA) * ydg) * yp
    inner_col = jnp.dot(p, ones, preferred_element_type=jnp.float32)
    w_col = jnp.log(s_col) + LAM * jnp.log(1.0 - inner_col)
    onesr = jnp.ones((1, w_col.shape[0]), jnp.float32)
    part = jnp.dot(onesr, w_col, preferred_element_type=jnp.float32)

    @pl.when(pl.program_id(0) == 0)
    def _():
        acc_ref[...] = jnp.zeros((1, 1), jnp.float32)
    acc_ref[...] += part

    @pl.when(pl.program_id(0) == pl.num_programs(0) - 1)
    def _():
        res_ref[...] = (acc_ref[...] - jnp.sum(pp_ref[...])) / B


def _tc_loss(output, qrows, outg, pick):
    B, C = output.shape
    G = B // BS
    return pl.pallas_call(
        functools.partial(_loss_body, B),
        grid=(G,),
        in_specs=[
            pl.BlockSpec((BS, C), lambda i: (i, 0)),
            pl.BlockSpec((BS, C), lambda i: (i, 0)),
            pl.BlockSpec((BS, C), lambda i: (i, 0)),
            pl.BlockSpec((NW, L), lambda i: (0, 0)),
        ],
        out_specs=pl.BlockSpec((1, 1), lambda i: (0, 0)),
        out_shape=jax.ShapeDtypeStruct((1, 1), jnp.float32),
        scratch_shapes=[pltpu.VMEM((1, 1), jnp.float32)],
    )(output, qrows, outg, pick)


def kernel(index, output, label, Q):
    B, C = output.shape
    N = Q.shape[0]
    rpw = B // NW
    vsp = ((N + NW - 1) // NW + 7) // 8 * 8

    idx = index.astype(jnp.int32)
    index3 = idx.reshape(NW, rpw // CH, CH)
    lab = label.astype(jnp.int32)
    outflat = output.reshape(B * C)

    t, qrows, pick = _sc_a(idx, lab, outflat, Q, vsp)
    outg = _sc_b(index3, t, output)
    res = _tc_loss(output, qrows, outg, pick)
    return res[0, 0]


# speedup vs baseline: 1.5047x; 1.0124x over previous
import functools

import jax
import jax.numpy as jnp
from jax import lax
from jax.experimental import pallas as pl
from jax.experimental.pallas import tpu as pltpu
from jax.experimental.pallas import tpu_sc as plsc

EMA = 0.7
LAM = 3.0
CLIP_LO = 0.0001
CLIP_HI = 1.0 - 0.0001

NC = 2
NS = 16
NW = NC * NS
L = 16
CH = 128
BS = 4096


def _wid():
    return lax.axis_index("s") * NC + lax.axis_index("c")


def _sc_a_body(B, C, vsp, idx_hbm, lab_hbm, outflat_hbm, q_hbm,
               t_hbm, qrows_hbm, pick_hbm,
               idx_all, t_loc, rows_v, lab_v, fidx_v, pick_v, acc_v,
               sem, sem2, sem3):
    wid = _wid()
    rpw = B // NW
    nk = rpw // CH
    lo = wid * vsp
    base = wid * rpw

    pltpu.sync_copy(idx_hbm, idx_all)
    pltpu.sync_copy(lab_hbm.at[pl.ds(base, rpw)], lab_v)

    qc = [pltpu.async_copy(
        q_hbm.at[idx_all.at[pl.ds(base + k * CH, CH)]],
        rows_v.at[pl.ds(k * CH, CH)], sem) for k in range(nk)]

    iota = lax.iota(jnp.int32, L)
    for k in range(rpw // L):
        row = (base + k * L) + iota
        fidx_v[pl.ds(k * L, L)] = row * C + lab_v[pl.ds(k * L, L)]
    pc = [pltpu.async_copy(
        outflat_hbm.at[fidx_v.at[pl.ds(k * CH, CH)]],
        pick_v.at[pl.ds(k * CH, CH)], sem3) for k in range(nk)]

    def step(k, carry):
        for u in range(8):
            kk = k * 8 + u
            iv = idx_all[pl.ds(kk * L, L)]
            jv = kk * L + iota
            owned = (iv >= lo) & (iv < lo + vsp)
            plsc.store_scatter(t_loc, [iv - lo], jv, mask=owned)
        return carry
    lax.fori_loop(0, B // L // 8, step, 0)
    pltpu.sync_copy(t_loc, t_hbm.at[pl.ds(lo, vsp)])

    for cp in qc:
        cp.wait()
    pltpu.sync_copy(rows_v, qrows_hbm.at[pl.ds(base, rpw)])

    for cp in pc:
        cp.wait()
    acc = pick_v[pl.ds(0, L)]
    for k in range(1, rpw // L):
        acc = acc + pick_v[pl.ds(k * L, L)]
    acc_v[...] = acc
    pltpu.sync_copy(acc_v, pick_hbm.at[wid])


def _sc_a(index, label, outflat, Q, vsp):
    B = index.shape[0]
    N, C = Q.shape
    rpw = B // NW
    mesh = plsc.VectorSubcoreMesh(core_axis_name="c", subcore_axis_name="s",
                                  num_cores=NC, num_subcores=NS)
    return pl.kernel(
        functools.partial(_sc_a_body, B, C, vsp),
        out_type=[
            jax.ShapeDtypeStruct((NW * vsp,), jnp.int32),
            jax.ShapeDtypeStruct((B, C), jnp.float32),
            jax.ShapeDtypeStruct((NW, L), jnp.float32),
        ],
        mesh=mesh,
        compiler_params=pltpu.CompilerParams(needs_layout_passes=False),
        scratch_types=[
            pltpu.VMEM((B,), jnp.int32),
            pltpu.VMEM((vsp,), jnp.int32),
            pltpu.VMEM((rpw, C), jnp.float32),
            pltpu.VMEM((rpw,), jnp.int32),
            pltpu.VMEM((rpw,), jnp.int32),
            pltpu.VMEM((rpw,), jnp.float32),
            pltpu.VMEM((L,), jnp.float32),
            pltpu.SemaphoreType.DMA,
            pltpu.SemaphoreType.DMA,
            pltpu.SemaphoreType.DMA,
        ],
    )(index, label, outflat, Q)


def _sc_b_body(B, idx3_hbm, t_hbm, src_hbm, outg_hbm, idx_v, jl_v, rows_v,
               semr, s0, s1, s2, s3):
    wid = _wid()
    rpw = B // NW
    nk = rpw // CH
    sems = [s0, s1, s2, s3]

    pltpu.sync_copy(idx3_hbm.at[wid], idx_v)
    jc = [pltpu.async_copy(t_hbm.at[idx_v.at[k]], jl_v.at[k], sems[k])
          for k in range(nk)]
    dc = []
    for k in range(nk):
        jc[k].wait()
        dc.append(pltpu.async_copy(src_hbm.at[jl_v.at[k]],
                                   rows_v.at[pl.ds(k * CH, CH)], semr))
    for cp in dc:
        cp.wait()
    pltpu.sync_copy(rows_v, outg_hbm.at[pl.ds(wid * rpw, rpw)])


def _sc_b(index3, t, src):
    B, C = src.shape
    rpw = B // NW
    mesh = plsc.VectorSubcoreMesh(core_axis_name="c", subcore_axis_name="s",
                                  num_cores=NC, num_subcores=NS)
    return pl.kernel(
        functools.partial(_sc_b_body, B),
        out_type=jax.ShapeDtypeStruct((B, C), jnp.float32),
        mesh=mesh,
        scratch_types=[
            pltpu.VMEM((rpw // CH, CH), jnp.int32),
            pltpu.VMEM((rpw // CH, CH), jnp.int32),
            pltpu.VMEM((rpw, C), jnp.float32),
            pltpu.SemaphoreType.DMA,
            pltpu.SemaphoreType.DMA,
            pltpu.SemaphoreType.DMA,
            pltpu.SemaphoreType.DMA,
            pltpu.SemaphoreType.DMA,
        ],
    )(index3, t, src)


def _loss_body(B, out_ref, qr_ref, og_ref, pp_ref, res_ref, acc_ref):
    x = out_ref[...]
    ex = jnp.exp(x)
    ones = jnp.ones((x.shape[1], 1), jnp.float32)
    s_col = jnp.dot(ex, ones, preferred_element_type=jnp.float32)
    yp = jnp.clip(ex * (1.0 / s_col), CLIP_LO, CLIP_HI)

    xg = og_ref[...]
    eg = jnp.exp(xg)
    sg_col = jnp.dot(eg, ones, preferred_element_type=jnp.float32)
    ypg = jnp.clip(eg * (1.0 / sg_col), CLIP_LO, CLIP_HI)
    spg_col = jnp.dot(ypg, ones, preferred_element_type=jnp.float32)
    ydg = ypg * (1.0 / spg_col)

    p = (EMA * qr_ref[...] + (1.0 - EMA) * ydg) * yp
    inner_col = jnp.dot(p, ones, preferred_element_type=jnp.float32)
    w_col = jnp.log(s_col) + LAM * jnp.log(1.0 - inner_col)
    onesr = jnp.ones((1, w_col.shape[0]), jnp.float32)
    part = jnp.dot(onesr, w_col, preferred_element_type=jnp.float32)

    @pl.when(pl.program_id(0) == 0)
    def _():
        acc_ref[...] = jnp.zeros((1, 1), jnp.float32)
    acc_ref[...] += part

    @pl.when(pl.program_id(0) == pl.num_programs(0) - 1)
    def _():
        res_ref[...] = (acc_ref[...] - jnp.sum(pp_ref[...])) / B


def _tc_loss(output, qrows, outg, pick):
    B, C = output.shape
    G = B // BS
    return pl.pallas_call(
        functools.partial(_loss_body, B),
        grid=(G,),
        in_specs=[
            pl.BlockSpec((BS, C), lambda i: (i, 0)),
            pl.BlockSpec((BS, C), lambda i: (i, 0)),
            pl.BlockSpec((BS, C), lambda i: (i, 0)),
            pl.BlockSpec((NW, L), lambda i: (0, 0)),
        ],
        out_specs=pl.BlockSpec((1, 1), lambda i: (0, 0)),
        out_shape=jax.ShapeDtypeStruct((1, 1), jnp.float32),
        scratch_shapes=[pltpu.VMEM((1, 1), jnp.float32)],
    )(output, qrows, outg, pick)


def kernel(index, output, label, Q):
    B, C = output.shape
    N = Q.shape[0]
    rpw = B // NW
    vsp = ((N + NW - 1) // NW + 7) // 8 * 8

    idx = index.astype(jnp.int32)
    index3 = idx.reshape(NW, rpw // CH, CH)
    lab = label.astype(jnp.int32)
    outflat = output.reshape(B * C)

    t, qrows, pick = _sc_a(idx, lab, outflat, Q, vsp)
    outg = _sc_b(index3, t, output)
    res = _tc_loss(output, qrows, outg, pick)
    return res[0, 0]
